# baseline (speedup 1.0000x reference)
import functools

import jax
import jax.numpy as jnp
from jax import lax
from jax.experimental import pallas as pl
from jax.experimental.pallas import tpu as pltpu
from jax.experimental.pallas import tpu_sc as plsc

_VOCAB = 1000000
_D = 16
_CB = 40000

_NC, _NS = 2, 16
_NW = _NC * _NS
_CHUNK = 2560


def _transform_body(t_ref, w_ref, b_ref, o_ref):
    x = t_ref[...]
    pid = pl.program_id(0)
    r = lax.broadcasted_iota(jnp.int32, x.shape, 0)
    c = lax.broadcasted_iota(jnp.int32, x.shape, 1)
    x = jnp.where((pid == 0) & (r == 0) & (c < _D), 0.0, x)
    o_ref[...] = (
        jnp.dot(x, w_ref[...], preferred_element_type=jnp.float32) + b_ref[...]
    )


def _transform_table(tblv, w128, b128):
    return pl.pallas_call(
        _transform_body,
        grid=(25,),
        in_specs=[
            pl.BlockSpec((5000, 128), lambda i: (i, 0)),
            pl.BlockSpec((128, 128), lambda i: (0, 0)),
            pl.BlockSpec((1, 128), lambda i: (0, 0)),
        ],
        out_specs=pl.BlockSpec((5000, 128), lambda i: (i, 0)),
        out_shape=jax.ShapeDtypeStruct((_VOCAB // 8, 128), jnp.float32),
    )(tblv, w128, b128)


_TCV = 1000


def _sc_transpose_in(tblT):
    mesh = plsc.VectorSubcoreMesh(core_axis_name="c", subcore_axis_name="s")

    @functools.partial(
        pl.kernel,
        mesh=mesh,
        compiler_params=pltpu.CompilerParams(
            use_tc_tiling_on_sc=False, needs_layout_passes=False
        ),
        out_type=jax.ShapeDtypeStruct((_VOCAB // 8, 128), jnp.float32),
        scratch_types=[
            pltpu.VMEM((_D, _TCV), jnp.float32),
            pltpu.VMEM((_TCV // 8, 128), jnp.float32),
            pltpu.SemaphoreType.DMA,
        ],
    )
    def k(t_hbm, o_hbm, in_v, out_v, sem):
        wid = lax.axis_index("s") * _NC + lax.axis_index("c")
        nj = 31 + jnp.where(wid < 8, 1, 0)
        g0 = wid * 31 + jnp.minimum(wid, 8)
        rows16 = lax.iota(jnp.int32, 16)

        @pl.loop(0, 32)
        def _(jl):
            @pl.when(jl < nj)
            def _():
                g = g0 + jl
                v0 = g * _TCV
                pltpu.sync_copy(t_hbm.at[:, pl.ds(v0, _TCV)], in_v)

                @pl.loop(0, _TCV)
                def _(v):
                    cols = jnp.full((16,), v, jnp.int32)
                    vec = plsc.load_gather(in_v, [rows16, cols])
                    out_v[v // 8, pl.ds((v % 8) * _D, _D)] = vec

                pltpu.sync_copy(out_v, o_hbm.at[pl.ds(g * (_TCV // 8), _TCV // 8)])

    return k(tblT)


def _sc_gather(table, idx):
    n = idx.shape[0]
    bpw = n // _NW
    nchunk = bpw // _CHUNK
    mesh = plsc.VectorSubcoreMesh(core_axis_name="c", subcore_axis_name="s")

    @functools.partial(
        pl.kernel,
        mesh=mesh,
        compiler_params=pltpu.CompilerParams(use_tc_tiling_on_sc=False),
        out_type=jax.ShapeDtypeStruct((n, _D), jnp.float32),
        scratch_types=[
            pltpu.VMEM((_CHUNK,), jnp.int32),
            pltpu.VMEM((_CHUNK, _D), jnp.float32),
            pltpu.SemaphoreType.DMA,
        ],
    )
    def k(table_hbm, idx_hbm, out_hbm, idx_v, rows_v, sem):
        wid = lax.axis_index("s") * _NC + lax.axis_index("c")
        base = wid * bpw

        @pl.loop(0, nchunk)
        def _(j):
            off = base + j * _CHUNK
            pltpu.sync_copy(idx_hbm.at[pl.ds(off, _CHUNK)], idx_v)
            pltpu.async_copy(table_hbm.at[idx_v], rows_v, sem).wait()
            pltpu.sync_copy(rows_v, out_hbm.at[pl.ds(off, _CHUNK)])

    return k(table, idx)


def kernel(x, emb_table, W, b):
    batch, hist = x.shape
    w128 = jnp.kron(jnp.eye(8, dtype=W.dtype), W.T)
    b128 = jnp.tile(b, 8).reshape(1, 128)
    tblv = _sc_transpose_in(emb_table.T)
    tbl_t = _transform_table(tblv, w128, b128)
    tbl_lin = tbl_t.reshape(_VOCAB * _D).reshape(_VOCAB, _D)
    idx = x.T.reshape(-1).astype(jnp.int32)
    out = _sc_gather(tbl_lin, idx)
    return jnp.transpose(out.reshape(hist, batch, _D), (1, 0, 2))


# speedup vs baseline: 2.1812x; 2.1812x over previous
import functools

import jax
import jax.numpy as jnp
from jax import lax
from jax.experimental import pallas as pl
from jax.experimental.pallas import tpu as pltpu
from jax.experimental.pallas import tpu_sc as plsc

_VOCAB = 1000000
_D = 16
_CB = 40000

_NC, _NS = 2, 16
_NW = _NC * _NS
_CHUNK = 2560


def _transform_body(t_ref, w_ref, b_ref, o_ref):
    x = t_ref[...]
    pid = pl.program_id(0)
    r = lax.broadcasted_iota(jnp.int32, x.shape, 0)
    c = lax.broadcasted_iota(jnp.int32, x.shape, 1)
    x = jnp.where((pid == 0) & (r == 0) & (c < _D), 0.0, x)
    o_ref[...] = (
        jnp.dot(x, w_ref[...], preferred_element_type=jnp.float32) + b_ref[...]
    )


def _transform_table(tblv, w128, b128):
    return pl.pallas_call(
        _transform_body,
        grid=(25,),
        in_specs=[
            pl.BlockSpec((5000, 128), lambda i: (i, 0)),
            pl.BlockSpec((128, 128), lambda i: (0, 0)),
            pl.BlockSpec((1, 128), lambda i: (0, 0)),
        ],
        out_specs=pl.BlockSpec((5000, 128), lambda i: (i, 0)),
        out_shape=jax.ShapeDtypeStruct((_VOCAB // 8, 128), jnp.float32),
    )(tblv, w128, b128)


def _sc_gather(table, idx):
    n = idx.shape[0]
    bpw = n // _NW
    nchunk = bpw // _CHUNK
    mesh = plsc.VectorSubcoreMesh(core_axis_name="c", subcore_axis_name="s")

    @functools.partial(
        pl.kernel,
        mesh=mesh,
        compiler_params=pltpu.CompilerParams(use_tc_tiling_on_sc=False),
        out_type=jax.ShapeDtypeStruct((n, _D), jnp.float32),
        scratch_types=[
            pltpu.VMEM((_CHUNK,), jnp.int32),
            pltpu.VMEM((_CHUNK, _D), jnp.float32),
            pltpu.SemaphoreType.DMA,
        ],
    )
    def k(table_hbm, idx_hbm, out_hbm, idx_v, rows_v, sem):
        wid = lax.axis_index("s") * _NC + lax.axis_index("c")
        base = wid * bpw

        @pl.loop(0, nchunk)
        def _(j):
            off = base + j * _CHUNK
            pltpu.sync_copy(idx_hbm.at[pl.ds(off, _CHUNK)], idx_v)
            pltpu.async_copy(table_hbm.at[idx_v], rows_v, sem).wait()
            pltpu.sync_copy(rows_v, out_hbm.at[pl.ds(off, _CHUNK)])

    return k(table, idx)


def kernel(x, emb_table, W, b):
    batch, hist = x.shape
    w128 = jnp.kron(jnp.eye(8, dtype=W.dtype), W.T)
    b128 = jnp.tile(b, 8).reshape(1, 128)
    tblv = emb_table.reshape(_VOCAB * _D).reshape(_VOCAB // 8, 128)
    tbl_t = _transform_table(tblv, w128, b128)
    tbl_lin = tbl_t.reshape(_VOCAB * _D).reshape(_VOCAB, _D)
    idx = x.T.reshape(-1).astype(jnp.int32)
    out = _sc_gather(tbl_lin, idx)
    return jnp.transpose(out.reshape(hist, batch, _D), (1, 0, 2))


# speedup vs baseline: 2.1983x; 1.0078x over previous
import functools

import jax
import jax.numpy as jnp
from jax import lax
from jax.experimental import pallas as pl
from jax.experimental.pallas import tpu as pltpu
from jax.experimental.pallas import tpu_sc as plsc

_VOCAB = 1000000
_D = 16
_CB = 40000

_NC, _NS = 2, 16
_NW = _NC * _NS
_CHUNK = 5120


def _transform_body(t_ref, w_ref, b_ref, o_ref):
    x = t_ref[...]
    pid = pl.program_id(0)
    r = lax.broadcasted_iota(jnp.int32, x.shape, 0)
    c = lax.broadcasted_iota(jnp.int32, x.shape, 1)
    x = jnp.where((pid == 0) & (r == 0) & (c < _D), 0.0, x)
    o_ref[...] = (
        jnp.dot(x, w_ref[...], preferred_element_type=jnp.float32) + b_ref[...]
    )


def _transform_table(tblv, w128, b128):
    return pl.pallas_call(
        _transform_body,
        grid=(25,),
        in_specs=[
            pl.BlockSpec((5000, 128), lambda i: (i, 0)),
            pl.BlockSpec((128, 128), lambda i: (0, 0)),
            pl.BlockSpec((1, 128), lambda i: (0, 0)),
        ],
        out_specs=pl.BlockSpec((5000, 128), lambda i: (i, 0)),
        out_shape=jax.ShapeDtypeStruct((_VOCAB // 8, 128), jnp.float32),
    )(tblv, w128, b128)


def _sc_gather(table, idx):
    n = idx.shape[0]
    bpw = n // _NW
    nchunk = bpw // _CHUNK
    mesh = plsc.VectorSubcoreMesh(core_axis_name="c", subcore_axis_name="s")

    @functools.partial(
        pl.kernel,
        mesh=mesh,
        compiler_params=pltpu.CompilerParams(use_tc_tiling_on_sc=False),
        out_type=jax.ShapeDtypeStruct((n, _D), jnp.float32),
        scratch_types=[
            pltpu.VMEM((_CHUNK,), jnp.int32),
            pltpu.VMEM((_CHUNK, _D), jnp.float32),
            pltpu.SemaphoreType.DMA,
        ],
    )
    def k(table_hbm, idx_hbm, out_hbm, idx_v, rows_v, sem):
        wid = lax.axis_index("s") * _NC + lax.axis_index("c")
        base = wid * bpw

        @pl.loop(0, nchunk)
        def _(j):
            off = base + j * _CHUNK
            pltpu.sync_copy(idx_hbm.at[pl.ds(off, _CHUNK)], idx_v)
            pltpu.async_copy(table_hbm.at[idx_v], rows_v, sem).wait()
            pltpu.sync_copy(rows_v, out_hbm.at[pl.ds(off, _CHUNK)])

    return k(table, idx)


def kernel(x, emb_table, W, b):
    batch, hist = x.shape
    w128 = jnp.kron(jnp.eye(8, dtype=W.dtype), W.T)
    b128 = jnp.tile(b, 8).reshape(1, 128)
    tblv = emb_table.reshape(_VOCAB * _D).reshape(_VOCAB // 8, 128)
    tbl_t = _transform_table(tblv, w128, b128)
    tbl_lin = tbl_t.reshape(_VOCAB * _D).reshape(_VOCAB, _D)
    idx = x.T.reshape(-1).astype(jnp.int32)
    out = _sc_gather(tbl_lin, idx)
    return jnp.transpose(out.reshape(hist, batch, _D), (1, 0, 2))
